# baseline (speedup 1.0000x reference)
import functools

import jax
import jax.numpy as jnp
from jax.experimental import pallas as pl
from jax.experimental.pallas import tpu as pltpu

BLK = 2048


def _fused_mlp_kernel(x_ref, cond_ref, w1_ref, wc_ref, b1_ref, w2_ref, b2_ref,
                      out_ref):
    x = x_ref[...].astype(jnp.bfloat16)
    c = cond_ref[...].astype(jnp.bfloat16)
    h = jnp.dot(x, w1_ref[...], preferred_element_type=jnp.float32)
    h += jnp.dot(c, wc_ref[...], preferred_element_type=jnp.float32)
    h += b1_ref[...]
    h = jnp.maximum(h, 0.0)
    out_ref[...] = (
        jnp.dot(h, w2_ref[...], preferred_element_type=jnp.float32)
        + b2_ref[...]
    )


@jax.jit
def kernel(x, cond, W1, Wc, b1, W2, b2):
    batch, input_dim = x.shape
    cond_dim = cond.shape[1]
    hidden = W1.shape[1]
    b1 = b1.reshape(1, hidden)
    b2 = b2.reshape(1, 1)
    W1 = W1.astype(jnp.bfloat16)
    Wc = Wc.astype(jnp.bfloat16)
    grid = (batch // BLK,)
    return pl.pallas_call(
        _fused_mlp_kernel,
        grid=grid,
        in_specs=[
            pl.BlockSpec((BLK, input_dim), lambda i: (i, 0)),
            pl.BlockSpec((BLK, cond_dim), lambda i: (i, 0)),
            pl.BlockSpec((input_dim, hidden), lambda i: (0, 0)),
            pl.BlockSpec((cond_dim, hidden), lambda i: (0, 0)),
            pl.BlockSpec((1, hidden), lambda i: (0, 0)),
            pl.BlockSpec((hidden, 1), lambda i: (0, 0)),
            pl.BlockSpec((1, 1), lambda i: (0, 0)),
        ],
        out_specs=pl.BlockSpec((BLK, 1), lambda i: (i, 0)),
        out_shape=jax.ShapeDtypeStruct((batch, 1), jnp.float32),
        compiler_params=pltpu.CompilerParams(
            dimension_semantics=("parallel",),
        ),
    )(x, cond, W1, Wc, b1, W2, b2)


# speedup vs baseline: 1.0652x; 1.0652x over previous
import functools

import jax
import jax.numpy as jnp
from jax.experimental import pallas as pl
from jax.experimental.pallas import tpu as pltpu

BLK = 4096


def _fused_mlp_kernel(x_ref, cond_ref, w1_ref, wc_ref, b1_ref, w2_ref, b2_ref,
                      out_ref):
    x = x_ref[...].astype(jnp.bfloat16)
    c = cond_ref[...].astype(jnp.bfloat16)
    h = jnp.dot(x, w1_ref[...], preferred_element_type=jnp.float32)
    h += jnp.dot(c, wc_ref[...], preferred_element_type=jnp.float32)
    h += b1_ref[...]
    h = jnp.maximum(h, 0.0)
    out_ref[...] = (
        jnp.dot(h, w2_ref[...], preferred_element_type=jnp.float32)
        + b2_ref[...]
    )


@jax.jit
def kernel(x, cond, W1, Wc, b1, W2, b2):
    batch, input_dim = x.shape
    cond_dim = cond.shape[1]
    hidden = W1.shape[1]
    b1 = b1.reshape(1, hidden)
    b2 = b2.reshape(1, 1)
    W1 = W1.astype(jnp.bfloat16)
    Wc = Wc.astype(jnp.bfloat16)
    grid = (batch // BLK,)
    return pl.pallas_call(
        _fused_mlp_kernel,
        grid=grid,
        in_specs=[
            pl.BlockSpec((BLK, input_dim), lambda i: (i, 0)),
            pl.BlockSpec((BLK, cond_dim), lambda i: (i, 0)),
            pl.BlockSpec((input_dim, hidden), lambda i: (0, 0)),
            pl.BlockSpec((cond_dim, hidden), lambda i: (0, 0)),
            pl.BlockSpec((1, hidden), lambda i: (0, 0)),
            pl.BlockSpec((hidden, 1), lambda i: (0, 0)),
            pl.BlockSpec((1, 1), lambda i: (0, 0)),
        ],
        out_specs=pl.BlockSpec((BLK, 1), lambda i: (i, 0)),
        out_shape=jax.ShapeDtypeStruct((batch, 1), jnp.float32),
        compiler_params=pltpu.CompilerParams(
            dimension_semantics=("parallel",),
        ),
    )(x, cond, W1, Wc, b1, W2, b2)


# speedup vs baseline: 1.1832x; 1.1108x over previous
import jax
import jax.numpy as jnp
from jax.experimental import pallas as pl
from jax.experimental.pallas import tpu as pltpu

BLK = 4096


def _fused_mlp_kernel(x_ref, cond_ref, w1_ref, wc_ref, b1_ref, w2_ref, b2_ref,
                      out_ref):
    x = x_ref[...].astype(jnp.bfloat16)
    c = cond_ref[...].astype(jnp.bfloat16)
    w1 = w1_ref[...].astype(jnp.bfloat16)
    wc = wc_ref[...].astype(jnp.bfloat16)
    h = jnp.dot(x, w1, preferred_element_type=jnp.float32)
    h += jnp.dot(c, wc, preferred_element_type=jnp.float32)
    h += b1_ref[...]
    h = jnp.maximum(h, 0.0)
    out_ref[...] = (
        jnp.dot(h, w2_ref[...], preferred_element_type=jnp.float32)
        + b2_ref[...]
    )


@jax.jit
def kernel(x, cond, W1, Wc, b1, W2, b2):
    batch, input_dim = x.shape
    cond_dim = cond.shape[1]
    hidden = W1.shape[1]
    grid = (batch // BLK,)
    return pl.pallas_call(
        _fused_mlp_kernel,
        grid=grid,
        in_specs=[
            pl.BlockSpec((BLK, input_dim), lambda i: (i, 0)),
            pl.BlockSpec((BLK, cond_dim), lambda i: (i, 0)),
            pl.BlockSpec((input_dim, hidden), lambda i: (0, 0)),
            pl.BlockSpec((cond_dim, hidden), lambda i: (0, 0)),
            pl.BlockSpec((hidden,), lambda i: (0,)),
            pl.BlockSpec((hidden, 1), lambda i: (0, 0)),
            pl.BlockSpec((1,), lambda i: (0,)),
        ],
        out_specs=pl.BlockSpec((BLK, 1), lambda i: (i, 0)),
        out_shape=jax.ShapeDtypeStruct((batch, 1), jnp.float32),
        compiler_params=pltpu.CompilerParams(
            dimension_semantics=("parallel",),
        ),
    )(x, cond, W1, Wc, b1, W2, b2)


# speedup vs baseline: 2.8889x; 2.4416x over previous
import jax
import jax.numpy as jnp
from jax import lax
from jax.experimental import pallas as pl
from jax.experimental.pallas import tpu as pltpu

BLK = 4096


def _fused_mlp_kernel(x_ref, ct_ref, w1_ref, wc_ref, b1_ref, w2_ref, b2_ref,
                      out_ref):
    x = x_ref[...].astype(jnp.bfloat16)
    ct = ct_ref[...].astype(jnp.bfloat16)
    w1 = w1_ref[...].astype(jnp.bfloat16)
    wc = wc_ref[...].astype(jnp.bfloat16)
    h = jnp.dot(x, w1, preferred_element_type=jnp.float32)
    h += lax.dot_general(ct, wc, (((0,), (0,)), ((), ())),
                         preferred_element_type=jnp.float32)
    h += b1_ref[...]
    h = jnp.maximum(h, 0.0)
    w2row = w2_ref[...][None, :]
    res = lax.dot_general(w2row, h, (((1,), (1,)), ((), ())),
                          preferred_element_type=jnp.float32)
    out_ref[...] = res[0] + b2_ref[0]


@jax.jit
def kernel(x, cond, W1, Wc, b1, W2, b2):
    batch, input_dim = x.shape
    cond_dim = cond.shape[1]
    hidden = W1.shape[1]
    ct = cond.T
    w2v = W2.reshape(hidden)
    grid = (batch // BLK,)
    out = pl.pallas_call(
        _fused_mlp_kernel,
        grid=grid,
        in_specs=[
            pl.BlockSpec((BLK, input_dim), lambda i: (i, 0)),
            pl.BlockSpec((cond_dim, BLK), lambda i: (0, i)),
            pl.BlockSpec((input_dim, hidden), lambda i: (0, 0)),
            pl.BlockSpec((cond_dim, hidden), lambda i: (0, 0)),
            pl.BlockSpec((hidden,), lambda i: (0,)),
            pl.BlockSpec((hidden,), lambda i: (0,)),
            pl.BlockSpec((1,), lambda i: (0,)),
        ],
        out_specs=pl.BlockSpec((BLK,), lambda i: (i,)),
        out_shape=jax.ShapeDtypeStruct((batch,), jnp.float32),
        compiler_params=pltpu.CompilerParams(
            dimension_semantics=("parallel",),
        ),
    )(x, ct, W1, Wc, b1, w2v, b2)
    return out.reshape(batch, 1)


# speedup vs baseline: 3.1322x; 1.0842x over previous
import jax
import jax.numpy as jnp
from jax import lax
from jax.experimental import pallas as pl
from jax.experimental.pallas import tpu as pltpu

BLK = 4096


def _fused_mlp_kernel(x_ref, ct_ref, w1_ref, wc_ref, b1_ref, w2_ref, b2_ref,
                      out_ref):
    x = x_ref[...].astype(jnp.bfloat16)
    ct = ct_ref[...].astype(jnp.bfloat16)
    w1 = w1_ref[...].astype(jnp.bfloat16)
    wc = wc_ref[...].astype(jnp.bfloat16)
    blk = ct.shape[1]
    cta = jnp.concatenate([ct, jnp.ones((1, blk), jnp.bfloat16)], axis=0)
    wca = jnp.concatenate([wc, b1_ref[...][None, :].astype(jnp.bfloat16)],
                          axis=0)
    h = jnp.dot(x, w1, preferred_element_type=jnp.float32)
    h += lax.dot_general(cta, wca, (((0,), (0,)), ((), ())),
                         preferred_element_type=jnp.float32)
    h = jnp.maximum(h, 0.0)
    w2f = w2_ref[...]
    q = (h[:, :128] * w2f[:128] + h[:, 128:] * w2f[128:]).astype(jnp.bfloat16)
    ones_row = jnp.ones((1, 128), jnp.bfloat16)
    res = lax.dot_general(ones_row, q, (((1,), (1,)), ((), ())),
                          preferred_element_type=jnp.float32)
    out_ref[...] = res[0] + b2_ref[0]


@jax.jit
def kernel(x, cond, W1, Wc, b1, W2, b2):
    batch, input_dim = x.shape
    cond_dim = cond.shape[1]
    hidden = W1.shape[1]
    ct = cond.T
    w2v = W2.reshape(hidden)
    grid = (batch // BLK,)
    out = pl.pallas_call(
        _fused_mlp_kernel,
        grid=grid,
        in_specs=[
            pl.BlockSpec((BLK, input_dim), lambda i: (i, 0)),
            pl.BlockSpec((cond_dim, BLK), lambda i: (0, i)),
            pl.BlockSpec((input_dim, hidden), lambda i: (0, 0)),
            pl.BlockSpec((cond_dim, hidden), lambda i: (0, 0)),
            pl.BlockSpec((hidden,), lambda i: (0,)),
            pl.BlockSpec((hidden,), lambda i: (0,)),
            pl.BlockSpec((1,), lambda i: (0,)),
        ],
        out_specs=pl.BlockSpec((BLK,), lambda i: (i,)),
        out_shape=jax.ShapeDtypeStruct((batch,), jnp.float32),
        compiler_params=pltpu.CompilerParams(
            dimension_semantics=("parallel",),
        ),
    )(x, ct, W1, Wc, b1, w2v, b2)
    return out.reshape(batch, 1)
